# Initial kernel scaffold; baseline (speedup 1.0000x reference)
import jax
import jax.numpy as jnp
from jax.experimental import pallas as pl


def kernel(loc_data, conf_data, loc_data2, conf_data2, dbox_list):
    raise NotImplementedError("write your pallas kernel here")


# speedup vs baseline: 376.8925x; 376.8925x over previous
import jax
import jax.numpy as jnp
from jax.experimental import pallas as pl
from jax.experimental.pallas import tpu as pltpu

_CONF = 0.01
_NMS = 0.45
_TOPK = 200
_NBOX = 20000
_HR = 160
_NR = 2 * _HR
_L = 128
_G = 20
_NBLK = _NR // 8
_R_KSC = 40
_R_KX1 = 42
_R_KY1 = 44
_R_KX2 = 46
_R_KY2 = 48
_R_KAR = 50


def _nms_kernel(s_ref, bx_ref, o_ref, s_scr, aux):
    ridx = jax.lax.broadcasted_iota(jnp.int32, (_NR, _L), 0)
    cidx = jax.lax.broadcasted_iota(jnp.int32, (_NR, _L), 1)
    pos = ((ridx % _HR) * _L + cidx)[None]

    kslot = (jax.lax.broadcasted_iota(jnp.int32, (1, 2, _L), 1) * _L
             + jax.lax.broadcasted_iota(jnp.int32, (1, 2, _L), 2))
    loc_idx = (jax.lax.broadcasted_iota(jnp.int32, (1, 8, _L), 1) * _L
               + jax.lax.broadcasted_iota(jnp.int32, (1, 8, _L), 2))
    blkiota = jax.lax.broadcasted_iota(jnp.int32, (1, _NBLK, _L), 1)

    p_raw = s_ref[0]
    masked = jnp.where((p_raw > _CONF) & (pos < _NBOX), p_raw, -1.0)
    s_scr[...] = masked
    any_m1 = jnp.any(masked[:, :_HR] > 0.0, axis=(1, 2), keepdims=True)
    aux[:, pl.ds(0, _NBLK), :] = jnp.max(
        masked.reshape(_G, _NBLK, 8, _L), axis=2)
    aux[:, pl.ds(_NBLK, 12), :] = jnp.zeros((_G, 12, _L), jnp.float32)

    def body(_, counts):
        bm = aux[:, pl.ds(0, _NBLK), :]
        m = jnp.max(bm, axis=(1, 2), keepdims=True)
        bsel = jnp.where(bm == m, blkiota, -1)
        bstar_v = jnp.max(bsel, axis=(1, 2), keepdims=True)
        blks = []
        bxs = []
        for g in range(_G):
            bstar = bstar_v[g, 0, 0]
            boff = pl.multiple_of(bstar * 8, 8)
            blks.append(s_scr[g, pl.ds(boff, 8), :])
            bxs.append(bx_ref[0, :, pl.ds(boff, 8), :])
        blk = jnp.stack(blks)
        bx = jnp.stack(bxs)
        t2 = jnp.where(blk == m, loc_idx, -1)
        p_loc = jnp.max(t2, axis=(1, 2), keepdims=True)
        onehot = t2 == p_loc
        newblk = jnp.where(onehot, -2.0, blk)
        for g in range(_G):
            bstar = bstar_v[g, 0, 0]
            boff = pl.multiple_of(bstar * 8, 8)
            s_scr[g, pl.ds(boff, 8), :] = newblk[g]
        newrow = jnp.max(newblk, axis=1, keepdims=True)
        for g in range(_G):
            bstar = bstar_v[g, 0, 0]
            aux[g, pl.ds(bstar, 1), :] = newrow[g]

        colv = jnp.max(jnp.where(onehot[:, None], bx, -jnp.inf), axis=2)
        ext = jnp.max(colv, axis=2, keepdims=True)[..., None]
        x1 = ext[:, 0]
        y1 = ext[:, 1]
        x2 = ext[:, 2]
        y2 = ext[:, 3]
        ar = (x2 - x1) * (y2 - y1)
        ksc = aux[:, pl.ds(_R_KSC, 2), :]
        kx1 = aux[:, pl.ds(_R_KX1, 2), :]
        ky1 = aux[:, pl.ds(_R_KY1, 2), :]
        kx2 = aux[:, pl.ds(_R_KX2, 2), :]
        ky2 = aux[:, pl.ds(_R_KY2, 2), :]
        kar = aux[:, pl.ds(_R_KAR, 2), :]
        xx1 = jnp.maximum(kx1, x1)
        yy1 = jnp.maximum(ky1, y1)
        xx2 = jnp.minimum(kx2, x2)
        yy2 = jnp.minimum(ky2, y2)
        w = jnp.maximum(xx2 - xx1, 0.0)
        h = jnp.maximum(yy2 - yy1, 0.0)
        inter = w * h
        union = (ar - inter) + kar
        iou = inter / union
        sup = jnp.logical_not(iou <= _NMS) & (kslot < counts)
        anysup = jnp.any(sup, axis=(1, 2), keepdims=True)
        take = (m > 0.0) & jnp.logical_not(anysup)
        sel = (kslot == counts) & take
        aux[:, pl.ds(_R_KSC, 2), :] = jnp.where(sel, m, ksc)
        aux[:, pl.ds(_R_KX1, 2), :] = jnp.where(sel, x1, kx1)
        aux[:, pl.ds(_R_KY1, 2), :] = jnp.where(sel, y1, ky1)
        aux[:, pl.ds(_R_KX2, 2), :] = jnp.where(sel, x2, kx2)
        aux[:, pl.ds(_R_KY2, 2), :] = jnp.where(sel, y2, ky2)
        aux[:, pl.ds(_R_KAR, 2), :] = jnp.where(sel, ar, kar)
        return counts + take.astype(jnp.int32)

    zc = jnp.zeros((_G, 1, 1), jnp.int32)
    jax.lax.fori_loop(0, _TOPK, body, zc, unroll=4)

    o_ref[0, :, pl.ds(0, 2), :] = jnp.where(any_m1, aux[:, pl.ds(_R_KSC, 2), :], 0.0)
    o_ref[0, :, pl.ds(2, 2), :] = jnp.where(any_m1, aux[:, pl.ds(_R_KX1, 2), :], 0.0)
    o_ref[0, :, pl.ds(4, 2), :] = jnp.where(any_m1, aux[:, pl.ds(_R_KY1, 2), :], 0.0)
    o_ref[0, :, pl.ds(6, 2), :] = jnp.where(any_m1, aux[:, pl.ds(_R_KX2, 2), :], 0.0)
    o_ref[0, :, pl.ds(8, 2), :] = jnp.where(any_m1, aux[:, pl.ds(_R_KY2, 2), :], 0.0)
    o_ref[0, :, pl.ds(10, 6), :] = jnp.zeros((_G, 6, _L), jnp.float32)


def _decode(loc, db):
    cxcy = db[None, :, :2] + loc[:, :, :2] * 0.1 * db[None, :, 2:]
    wh = db[None, :, 2:] * jnp.exp(loc[:, :, 2:] * 0.2)
    mins = cxcy - wh / 2.0
    maxs = mins + wh
    return jnp.concatenate([mins, maxs], axis=-1)


def _plane(x):
    xp = jnp.pad(x, ((0, 0), (0, _HR * _L - _NBOX)))
    return xp.reshape(2, _HR, _L)


def _class_planes(sm):
    smt = jnp.transpose(sm, (0, 2, 1))
    smp = jnp.pad(smt, ((0, 0), (0, 0), (0, _HR * _L - _NBOX)))
    return smp.reshape(2, 21, _HR, _L)


def kernel(loc_data, conf_data, loc_data2, conf_data2, dbox_list):
    sm1 = jax.nn.softmax(conf_data, axis=-1)
    sm2 = jax.nn.softmax(conf_data2, axis=-1)
    d1 = _decode(loc_data, dbox_list)
    d2 = _decode(loc_data2, dbox_list)
    b2 = jnp.concatenate([1.0 - d2[:, :, 2:3], d2[:, :, 1:]], axis=-1)

    scores = jnp.concatenate([_class_planes(sm1), _class_planes(sm2)], axis=2)
    scores = scores[:, 1:21]
    bx1 = jnp.stack([_plane(d1[..., i]) for i in range(4)], axis=1)
    bx2 = jnp.stack([_plane(b2[..., i]) for i in range(4)], axis=1)
    bxs = jnp.concatenate([bx1, bx2], axis=2)

    out = pl.pallas_call(
        _nms_kernel,
        grid=(2, 20 // _G),
        in_specs=[
            pl.BlockSpec((1, _G, _NR, _L), lambda b, c: (b, c, 0, 0)),
            pl.BlockSpec((1, 4, _NR, _L), lambda b, c: (b, 0, 0, 0)),
        ],
        out_specs=pl.BlockSpec((1, _G, 16, _L), lambda b, c: (b, c, 0, 0)),
        out_shape=jax.ShapeDtypeStruct((2, 20, 16, _L), jnp.float32),
        scratch_shapes=[pltpu.VMEM((_G, _NR, _L), jnp.float32),
                        pltpu.VMEM((_G, 52, _L), jnp.float32)],
    )(scores, bxs)

    rows = out[:, :, :10, :].reshape(2, 20, 5, 2 * _L)
    rows = jnp.transpose(rows, (0, 1, 3, 2))[:, :, :_TOPK, :]
    bg = jnp.zeros((2, 1, _TOPK, 5), jnp.float32)
    return jnp.concatenate([bg, rows], axis=1)
